# Initial kernel scaffold; baseline (speedup 1.0000x reference)
import jax
import jax.numpy as jnp
from jax.experimental import pallas as pl


def kernel(n_feat, edge_index, W1, b1, W2, b2, W_fc, b_fc):
    raise NotImplementedError("write your pallas kernel here")


# speedup vs baseline: 3.8594x; 3.8594x over previous
import jax
import jax.numpy as jnp
from jax import lax
from jax.experimental import pallas as pl
from jax.experimental.pallas import tpu as pltpu
from jax.experimental.pallas import tpu_sc as plsc

N = 10000
NP = 10240
E = 40000
PATCH = 84
D = 32
FEAT = PATCH * D

NW = 32
ROWS_PW = FEAT // NW
FPP = 4
PASSES = ROWS_PW // FPP


def _vmesh():
    return plsc.VectorSubcoreMesh(core_axis_name="c", subcore_axis_name="s",
                                  num_cores=2, num_subcores=16)


def _prep_body(edge_ref, ns_ref, nd_ref, pk_ref, esrc, sbuf, dbuf, pbuf, hist):
    wid = lax.axis_index("s") * 2 + lax.axis_index("c")
    ones = jnp.full((16,), 1.0, jnp.float32)

    def hist_norm(row, out_hbm):
        pltpu.sync_copy(edge_ref.at[pl.ds(row * E, E)], esrc)

        def zero(i, _):
            hist[pl.ds(i * 16, 16)] = jnp.zeros((16,), jnp.float32)
            return 0
        lax.fori_loop(0, NP // 16, zero, 0)

        def acc(g, _):
            idx = esrc[pl.ds(g * 16, 16)]
            plsc.addupdate_scatter(hist, [idx], ones)
            return 0
        lax.fori_loop(0, E // 16, acc, 0)

        def norm(i, _):
            deg = hist[pl.ds(i * 16, 16)]
            m = jnp.maximum(deg, 1.0)
            bi = plsc.bitcast(m, jnp.int32)
            bi = jnp.int32(0x5F3759DF) - (bi >> 1)
            y = plsc.bitcast(bi, jnp.float32)
            for _ in range(4):
                y = y * (1.5 - 0.5 * m * y * y)
            hist[pl.ds(i * 16, 16)] = y
            return 0
        lax.fori_loop(0, NP // 16, norm, 0)
        pltpu.sync_copy(hist, out_hbm)

    @pl.when(wid == 0)
    def _():
        hist_norm(0, ns_ref)

    @pl.when(wid == 1)
    def _():
        hist_norm(1, nd_ref)

    @pl.when((wid >= 2) & (wid < 27))
    def _():
        off = (wid - 2) * (E // 25)
        pltpu.sync_copy(edge_ref.at[pl.ds(off, E // 25)], sbuf)
        pltpu.sync_copy(edge_ref.at[pl.ds(E + off, E // 25)], dbuf)

        def pk(g, _):
            s = sbuf[pl.ds(g * 16, 16)]
            d = dbuf[pl.ds(g * 16, 16)]
            pbuf[pl.ds(g * 16, 16)] = s | (d << 14)
            return 0
        lax.fori_loop(0, (E // 25) // 16, pk, 0)
        pltpu.sync_copy(pbuf, pk_ref.at[pl.ds(off, E // 25)])


_prep = pl.kernel(
    _prep_body,
    out_type=[
        jax.ShapeDtypeStruct((NP,), jnp.float32),
        jax.ShapeDtypeStruct((NP,), jnp.float32),
        jax.ShapeDtypeStruct((E,), jnp.int32),
    ],
    mesh=_vmesh(),
    scratch_types=[
        pltpu.VMEM((E,), jnp.int32),
        pltpu.VMEM((E // 25,), jnp.int32),
        pltpu.VMEM((E // 25,), jnp.int32),
        pltpu.VMEM((E // 25,), jnp.int32),
        pltpu.VMEM((NP,), jnp.float32),
    ],
    compiler_params=pltpu.CompilerParams(needs_layout_passes=False),
)


def _spmm_body(xt_ref, pk_ref, yt_ref, ebuf, x_buf, y_buf):
    wid = lax.axis_index("s") * 2 + lax.axis_index("c")
    pltpu.sync_copy(pk_ref, ebuf)
    base = wid * ROWS_PW
    mask14 = jnp.full((16,), 0x3FFF, jnp.int32)

    def one_pass(p, _):
        row0 = base + p * FPP
        pltpu.sync_copy(xt_ref.at[pl.ds(row0, FPP)], x_buf)

        def zero(i, _):
            for r in range(FPP):
                y_buf[r, pl.ds(i * 16, 16)] = jnp.zeros((16,), jnp.float32)
            return 0
        lax.fori_loop(0, NP // 16, zero, 0)

        def edge(g, _):
            pe = ebuf[pl.ds(g * 16, 16)]
            s = pe & mask14
            d = lax.shift_right_logical(pe, 14)
            for r in range(FPP):
                ridx = jnp.full((16,), r, jnp.int32)
                v = plsc.load_gather(x_buf, [ridx, s])
                plsc.addupdate_scatter(y_buf, [ridx, d], v)
            return 0
        lax.fori_loop(0, E // 16, edge, 0)
        pltpu.sync_copy(y_buf, yt_ref.at[pl.ds(row0, FPP)])
        return 0
    lax.fori_loop(0, PASSES, one_pass, 0)


_spmm = pl.kernel(
    _spmm_body,
    out_type=jax.ShapeDtypeStruct((FEAT, NP), jnp.float32),
    mesh=_vmesh(),
    scratch_types=[
        pltpu.VMEM((E,), jnp.int32),
        pltpu.VMEM((FPP, NP), jnp.float32),
        pltpu.VMEM((FPP, NP), jnp.float32),
    ],
    compiler_params=pltpu.CompilerParams(needs_layout_passes=False),
)


def _tr_body(x_ref, ns_ref, o_ref):
    o_ref[...] = jnp.transpose(x_ref[...]) * ns_ref[...]


def _transpose_scale(x0, ns2):
    blk = 512
    return pl.pallas_call(
        _tr_body,
        grid=(NP // blk,),
        in_specs=[
            pl.BlockSpec((blk, FEAT), lambda g: (g, 0)),
            pl.BlockSpec((1, blk), lambda g: (0, g)),
        ],
        out_specs=pl.BlockSpec((FEAT, blk), lambda g: (0, g)),
        out_shape=jax.ShapeDtypeStruct((FEAT, NP), jnp.float32),
    )(x0, ns2)


def _d1_body(y_ref, nd_ref, ns_ref, w_ref, b_ref, o_ref):
    y = y_ref[...] * nd_ref[...]
    h = lax.dot_general(w_ref[...], y, (((0,), (0,)), ((), ())),
                        preferred_element_type=jnp.float32,
                        precision=lax.Precision.HIGHEST)
    h = h + b_ref[...]
    h = jnp.where(h >= 0, h, 0.01 * h)
    o_ref[...] = h * ns_ref[...]


def _mid_layer(y1, nd2, ns2, W1, b1c):
    return pl.pallas_call(
        _d1_body,
        grid=(PATCH,),
        in_specs=[
            pl.BlockSpec((D, NP), lambda p: (p, 0)),
            pl.BlockSpec((1, NP), lambda p: (0, 0)),
            pl.BlockSpec((1, NP), lambda p: (0, 0)),
            pl.BlockSpec((D, D), lambda p: (0, 0)),
            pl.BlockSpec((D, 1), lambda p: (0, 0)),
        ],
        out_specs=pl.BlockSpec((D, NP), lambda p: (p, 0)),
        out_shape=jax.ShapeDtypeStruct((FEAT, NP), jnp.float32),
    )(y1, nd2, ns2, W1, b1c)


def _d2_body(y_ref, nd_ref, w2_ref, b2_ref, wfc_ref, bfc_ref, o_ref):
    p = pl.program_id(0)
    y = y_ref[...] * nd_ref[...]
    h = lax.dot_general(w2_ref[...], y, (((0,), (0,)), ((), ())),
                        preferred_element_type=jnp.float32,
                        precision=lax.Precision.HIGHEST)
    h = h + b2_ref[...]
    h = jnp.where(h >= 0, h, 0.01 * h)
    mask = lax.broadcasted_iota(jnp.int32, (D, NP), 1) < N
    h = jnp.where(mask, h, 0.0)
    s = jnp.sum(h, axis=1, keepdims=True)
    contrib = lax.dot_general(s, wfc_ref[...], (((0,), (0,)), ((), ())),
                              preferred_element_type=jnp.float32,
                              precision=lax.Precision.HIGHEST)

    @pl.when(p == 0)
    def _():
        o_ref[...] = bfc_ref[...]

    o_ref[...] += contrib * (1.0 / N)


def _final_layer(y2, nd2, W2, b2c, W_fc_t, bfc2):
    return pl.pallas_call(
        _d2_body,
        grid=(PATCH,),
        in_specs=[
            pl.BlockSpec((D, NP), lambda p: (p, 0)),
            pl.BlockSpec((1, NP), lambda p: (0, 0)),
            pl.BlockSpec((D, D), lambda p: (0, 0)),
            pl.BlockSpec((D, 1), lambda p: (0, 0)),
            pl.BlockSpec((D, D), lambda p: (p, 0)),
            pl.BlockSpec((1, D), lambda p: (0, 0)),
        ],
        out_specs=pl.BlockSpec((1, D), lambda p: (0, 0)),
        out_shape=jax.ShapeDtypeStruct((1, D), jnp.float32),
    )(y2, nd2, W2, b2c, W_fc_t, bfc2)


def kernel(n_feat, edge_index, W1, b1, W2, b2, W_fc, b_fc):
    x0 = n_feat.reshape(N, FEAT)
    ns, nd, pk = _prep(edge_index.reshape(2 * E))
    ns2 = ns.reshape(1, NP)
    nd2 = nd.reshape(1, NP)
    xt = _transpose_scale(x0, ns2)
    y1 = _spmm(xt, pk)
    z1 = _mid_layer(y1, nd2, ns2, W1, b1.reshape(D, 1))
    y2 = _spmm(z1, pk)
    return _final_layer(y2, nd2, W2, b2.reshape(D, 1), W_fc.T, b_fc.reshape(1, D))


# speedup vs baseline: 8.3876x; 2.1733x over previous
import jax
import jax.numpy as jnp
from jax import lax
from jax.experimental import pallas as pl
from jax.experimental.pallas import tpu as pltpu
from jax.experimental.pallas import tpu_sc as plsc

N = 10000
NP = 10240
E = 40000
PATCH = 84
D = 32
FEAT = PATCH * D

NW = 32
ROWS_PW = FEAT // NW
FPP = 4
PASSES = ROWS_PW // FPP


def _vmesh():
    return plsc.VectorSubcoreMesh(core_axis_name="c", subcore_axis_name="s",
                                  num_cores=2, num_subcores=16)


def _prep_body(edge_ref, ns_ref, nd_ref, pk_ref, esrc, sbuf, dbuf, pbuf, hist):
    wid = lax.axis_index("s") * 2 + lax.axis_index("c")
    ones = jnp.full((16,), 1.0, jnp.float32)

    def hist_norm(row, out_hbm):
        pltpu.sync_copy(edge_ref.at[pl.ds(row * E, E)], esrc)

        def zero(i, _):
            hist[pl.ds(i * 16, 16)] = jnp.zeros((16,), jnp.float32)
            return 0
        lax.fori_loop(0, NP // 16, zero, 0)

        def acc(g, _):
            idx = esrc[pl.ds(g * 16, 16)]
            plsc.addupdate_scatter(hist, [idx], ones)
            return 0
        lax.fori_loop(0, E // 16, acc, 0)

        def norm(i, _):
            deg = hist[pl.ds(i * 16, 16)]
            m = jnp.maximum(deg, 1.0)
            bi = plsc.bitcast(m, jnp.int32)
            bi = jnp.int32(0x5F3759DF) - (bi >> 1)
            y = plsc.bitcast(bi, jnp.float32)
            for _ in range(4):
                y = y * (1.5 - 0.5 * m * y * y)
            hist[pl.ds(i * 16, 16)] = y
            return 0
        lax.fori_loop(0, NP // 16, norm, 0)
        pltpu.sync_copy(hist, out_hbm)

    @pl.when(wid == 0)
    def _():
        hist_norm(0, ns_ref)

    @pl.when(wid == 1)
    def _():
        hist_norm(1, nd_ref)

    @pl.when((wid >= 2) & (wid < 27))
    def _():
        off = (wid - 2) * (E // 25)
        pltpu.sync_copy(edge_ref.at[pl.ds(off, E // 25)], sbuf)
        pltpu.sync_copy(edge_ref.at[pl.ds(E + off, E // 25)], dbuf)

        def pk(g, _):
            s = sbuf[pl.ds(g * 16, 16)]
            d = dbuf[pl.ds(g * 16, 16)]
            pbuf[pl.ds(g * 16, 16)] = s | (d << 14)
            return 0
        lax.fori_loop(0, (E // 25) // 16, pk, 0)
        pltpu.sync_copy(pbuf, pk_ref.at[pl.ds(off, E // 25)])


_prep = pl.kernel(
    _prep_body,
    out_type=[
        jax.ShapeDtypeStruct((NP,), jnp.float32),
        jax.ShapeDtypeStruct((NP,), jnp.float32),
        jax.ShapeDtypeStruct((E,), jnp.int32),
    ],
    mesh=_vmesh(),
    scratch_types=[
        pltpu.VMEM((E,), jnp.int32),
        pltpu.VMEM((E // 25,), jnp.int32),
        pltpu.VMEM((E // 25,), jnp.int32),
        pltpu.VMEM((E // 25,), jnp.int32),
        pltpu.VMEM((NP,), jnp.float32),
    ],
    compiler_params=pltpu.CompilerParams(needs_layout_passes=False),
)


def _spmm_body(xt_ref, pk_ref, yt_ref, ebuf, x_buf, y_buf):
    wid = lax.axis_index("s") * 2 + lax.axis_index("c")
    pltpu.sync_copy(pk_ref, ebuf)
    base = wid * ROWS_PW
    mask14 = jnp.full((16,), 0x3FFF, jnp.int32)

    def one_pass(p, _):
        row0 = base + p * FPP
        pltpu.sync_copy(xt_ref.at[pl.ds(row0, FPP)], x_buf)

        @plsc.parallel_loop(0, NP // 16)
        def _zero(i):
            for r in range(FPP):
                y_buf[r, pl.ds(i * 16, 16)] = jnp.zeros((16,), jnp.float32)

        @plsc.parallel_loop(0, E // 16, unroll=4)
        def _edge(g):
            pe = ebuf[pl.ds(g * 16, 16)]
            s = pe & mask14
            d = lax.shift_right_logical(pe, 14)
            for r in range(FPP):
                ridx = jnp.full((16,), r, jnp.int32)
                v = plsc.load_gather(x_buf, [ridx, s])
                plsc.addupdate_scatter(y_buf, [ridx, d], v)

        pltpu.sync_copy(y_buf, yt_ref.at[pl.ds(row0, FPP)])
        return 0
    lax.fori_loop(0, PASSES, one_pass, 0)


_spmm = pl.kernel(
    _spmm_body,
    out_type=jax.ShapeDtypeStruct((FEAT, NP), jnp.float32),
    mesh=_vmesh(),
    scratch_types=[
        pltpu.VMEM((E,), jnp.int32),
        pltpu.VMEM((FPP, NP), jnp.float32),
        pltpu.VMEM((FPP, NP), jnp.float32),
    ],
    compiler_params=pltpu.CompilerParams(needs_layout_passes=False),
)


def _tr_body(x_ref, ns_ref, o_ref):
    o_ref[...] = jnp.transpose(x_ref[...]) * ns_ref[...]


def _transpose_scale(x0, ns2):
    blk = 512
    return pl.pallas_call(
        _tr_body,
        grid=(NP // blk,),
        in_specs=[
            pl.BlockSpec((blk, FEAT), lambda g: (g, 0)),
            pl.BlockSpec((1, blk), lambda g: (0, g)),
        ],
        out_specs=pl.BlockSpec((FEAT, blk), lambda g: (0, g)),
        out_shape=jax.ShapeDtypeStruct((FEAT, NP), jnp.float32),
    )(x0, ns2)


def _d1_body(y_ref, nd_ref, ns_ref, w_ref, b_ref, o_ref):
    y = y_ref[...] * nd_ref[...]
    h = lax.dot_general(w_ref[...], y, (((0,), (0,)), ((), ())),
                        preferred_element_type=jnp.float32,
                        precision=lax.Precision.HIGHEST)
    h = h + b_ref[...]
    h = jnp.where(h >= 0, h, 0.01 * h)
    o_ref[...] = h * ns_ref[...]


def _mid_layer(y1, nd2, ns2, W1, b1c):
    return pl.pallas_call(
        _d1_body,
        grid=(PATCH,),
        in_specs=[
            pl.BlockSpec((D, NP), lambda p: (p, 0)),
            pl.BlockSpec((1, NP), lambda p: (0, 0)),
            pl.BlockSpec((1, NP), lambda p: (0, 0)),
            pl.BlockSpec((D, D), lambda p: (0, 0)),
            pl.BlockSpec((D, 1), lambda p: (0, 0)),
        ],
        out_specs=pl.BlockSpec((D, NP), lambda p: (p, 0)),
        out_shape=jax.ShapeDtypeStruct((FEAT, NP), jnp.float32),
    )(y1, nd2, ns2, W1, b1c)


def _d2_body(y_ref, nd_ref, w2_ref, b2_ref, wfc_ref, bfc_ref, o_ref):
    p = pl.program_id(0)
    y = y_ref[...] * nd_ref[...]
    h = lax.dot_general(w2_ref[...], y, (((0,), (0,)), ((), ())),
                        preferred_element_type=jnp.float32,
                        precision=lax.Precision.HIGHEST)
    h = h + b2_ref[...]
    h = jnp.where(h >= 0, h, 0.01 * h)
    mask = lax.broadcasted_iota(jnp.int32, (D, NP), 1) < N
    h = jnp.where(mask, h, 0.0)
    s = jnp.sum(h, axis=1, keepdims=True)
    contrib = lax.dot_general(s, wfc_ref[...], (((0,), (0,)), ((), ())),
                              preferred_element_type=jnp.float32,
                              precision=lax.Precision.HIGHEST)

    @pl.when(p == 0)
    def _():
        o_ref[...] = bfc_ref[...]

    o_ref[...] += contrib * (1.0 / N)


def _final_layer(y2, nd2, W2, b2c, W_fc_t, bfc2):
    return pl.pallas_call(
        _d2_body,
        grid=(PATCH,),
        in_specs=[
            pl.BlockSpec((D, NP), lambda p: (p, 0)),
            pl.BlockSpec((1, NP), lambda p: (0, 0)),
            pl.BlockSpec((D, D), lambda p: (0, 0)),
            pl.BlockSpec((D, 1), lambda p: (0, 0)),
            pl.BlockSpec((D, D), lambda p: (p, 0)),
            pl.BlockSpec((1, D), lambda p: (0, 0)),
        ],
        out_specs=pl.BlockSpec((1, D), lambda p: (0, 0)),
        out_shape=jax.ShapeDtypeStruct((1, D), jnp.float32),
    )(y2, nd2, W2, b2c, W_fc_t, bfc2)


def kernel(n_feat, edge_index, W1, b1, W2, b2, W_fc, b_fc):
    x0 = n_feat.reshape(N, FEAT)
    ns, nd, pk = _prep(edge_index.reshape(2 * E))
    ns2 = ns.reshape(1, NP)
    nd2 = nd.reshape(1, NP)
    xt = _transpose_scale(x0, ns2)
    y1 = _spmm(xt, pk)
    z1 = _mid_layer(y1, nd2, ns2, W1, b1.reshape(D, 1))
    y2 = _spmm(z1, pk)
    return _final_layer(y2, nd2, W2, b2.reshape(D, 1), W_fc.T, b_fc.reshape(1, D))


# speedup vs baseline: 8.4264x; 1.0046x over previous
import jax
import jax.numpy as jnp
from jax import lax
from jax.experimental import pallas as pl
from jax.experimental.pallas import tpu as pltpu
from jax.experimental.pallas import tpu_sc as plsc

N = 10000
NP = 10240
E = 40000
PATCH = 84
D = 32
FEAT = PATCH * D

NW = 32
ROWS_PW = FEAT // NW
FPP = 4
PASSES = ROWS_PW // FPP


def _vmesh():
    return plsc.VectorSubcoreMesh(core_axis_name="c", subcore_axis_name="s",
                                  num_cores=2, num_subcores=16)


def _prep_body(edge_ref, ns_ref, nd_ref, pk_ref, esrc, sbuf, dbuf, pbuf, hist):
    wid = lax.axis_index("s") * 2 + lax.axis_index("c")
    ones = jnp.full((16,), 1.0, jnp.float32)

    def hist_norm(row, out_hbm):
        pltpu.sync_copy(edge_ref.at[pl.ds(row * E, E)], esrc)

        def zero(i, _):
            hist[pl.ds(i * 16, 16)] = jnp.zeros((16,), jnp.float32)
            return 0
        lax.fori_loop(0, NP // 16, zero, 0)

        def acc(g, _):
            idx = esrc[pl.ds(g * 16, 16)]
            plsc.addupdate_scatter(hist, [idx], ones)
            return 0
        lax.fori_loop(0, E // 16, acc, 0)

        def norm(i, _):
            deg = hist[pl.ds(i * 16, 16)]
            m = jnp.maximum(deg, 1.0)
            bi = plsc.bitcast(m, jnp.int32)
            bi = jnp.int32(0x5F3759DF) - (bi >> 1)
            y = plsc.bitcast(bi, jnp.float32)
            for _ in range(4):
                y = y * (1.5 - 0.5 * m * y * y)
            hist[pl.ds(i * 16, 16)] = y
            return 0
        lax.fori_loop(0, NP // 16, norm, 0)
        pltpu.sync_copy(hist, out_hbm)

    @pl.when(wid == 0)
    def _():
        hist_norm(0, ns_ref)

    @pl.when(wid == 1)
    def _():
        hist_norm(1, nd_ref)

    @pl.when((wid >= 2) & (wid < 27))
    def _():
        off = (wid - 2) * (E // 25)
        pltpu.sync_copy(edge_ref.at[pl.ds(off, E // 25)], sbuf)
        pltpu.sync_copy(edge_ref.at[pl.ds(E + off, E // 25)], dbuf)

        def pk(g, _):
            s = sbuf[pl.ds(g * 16, 16)]
            d = dbuf[pl.ds(g * 16, 16)]
            pbuf[pl.ds(g * 16, 16)] = s | (d << 14)
            return 0
        lax.fori_loop(0, (E // 25) // 16, pk, 0)
        pltpu.sync_copy(pbuf, pk_ref.at[pl.ds(off, E // 25)])


_prep = pl.kernel(
    _prep_body,
    out_type=[
        jax.ShapeDtypeStruct((NP,), jnp.float32),
        jax.ShapeDtypeStruct((NP,), jnp.float32),
        jax.ShapeDtypeStruct((E,), jnp.int32),
    ],
    mesh=_vmesh(),
    scratch_types=[
        pltpu.VMEM((E,), jnp.int32),
        pltpu.VMEM((E // 25,), jnp.int32),
        pltpu.VMEM((E // 25,), jnp.int32),
        pltpu.VMEM((E // 25,), jnp.int32),
        pltpu.VMEM((NP,), jnp.float32),
    ],
    compiler_params=pltpu.CompilerParams(needs_layout_passes=False),
)


def _spmm_body(xt_ref, pk_ref, yt_ref, ebuf, x_buf, y_buf):
    wid = lax.axis_index("s") * 2 + lax.axis_index("c")
    pltpu.sync_copy(pk_ref, ebuf)
    base = wid * ROWS_PW
    mask14 = jnp.full((16,), 0x3FFF, jnp.int32)

    def one_pass(p, _):
        row0 = base + p * FPP
        pltpu.sync_copy(xt_ref.at[pl.ds(row0, FPP)], x_buf)

        @plsc.parallel_loop(0, NP // 16)
        def _zero(i):
            for r in range(FPP):
                y_buf[r, pl.ds(i * 16, 16)] = jnp.zeros((16,), jnp.float32)

        @plsc.parallel_loop(0, E // 16, unroll=8)
        def _edge(g):
            pe = ebuf[pl.ds(g * 16, 16)]
            s = pe & mask14
            d = lax.shift_right_logical(pe, 14)
            for r in range(FPP):
                ridx = jnp.full((16,), r, jnp.int32)
                v = plsc.load_gather(x_buf, [ridx, s])
                plsc.addupdate_scatter(y_buf, [ridx, d], v)

        pltpu.sync_copy(y_buf, yt_ref.at[pl.ds(row0, FPP)])
        return 0
    lax.fori_loop(0, PASSES, one_pass, 0)


_spmm = pl.kernel(
    _spmm_body,
    out_type=jax.ShapeDtypeStruct((FEAT, NP), jnp.float32),
    mesh=_vmesh(),
    scratch_types=[
        pltpu.VMEM((E,), jnp.int32),
        pltpu.VMEM((FPP, NP), jnp.float32),
        pltpu.VMEM((FPP, NP), jnp.float32),
    ],
    compiler_params=pltpu.CompilerParams(needs_layout_passes=False),
)


def _tr_body(x_ref, ns_ref, o_ref):
    o_ref[...] = jnp.transpose(x_ref[...]) * ns_ref[...]


def _transpose_scale(x0, ns2):
    blk = 512
    return pl.pallas_call(
        _tr_body,
        grid=(NP // blk,),
        in_specs=[
            pl.BlockSpec((blk, FEAT), lambda g: (g, 0)),
            pl.BlockSpec((1, blk), lambda g: (0, g)),
        ],
        out_specs=pl.BlockSpec((FEAT, blk), lambda g: (0, g)),
        out_shape=jax.ShapeDtypeStruct((FEAT, NP), jnp.float32),
    )(x0, ns2)


def _d1_body(y_ref, nd_ref, ns_ref, w_ref, b_ref, o_ref):
    y = y_ref[...] * nd_ref[...]
    h = lax.dot_general(w_ref[...], y, (((0,), (0,)), ((), ())),
                        preferred_element_type=jnp.float32,
                        precision=lax.Precision.HIGHEST)
    h = h + b_ref[...]
    h = jnp.where(h >= 0, h, 0.01 * h)
    o_ref[...] = h * ns_ref[...]


def _mid_layer(y1, nd2, ns2, W1, b1c):
    return pl.pallas_call(
        _d1_body,
        grid=(PATCH,),
        in_specs=[
            pl.BlockSpec((D, NP), lambda p: (p, 0)),
            pl.BlockSpec((1, NP), lambda p: (0, 0)),
            pl.BlockSpec((1, NP), lambda p: (0, 0)),
            pl.BlockSpec((D, D), lambda p: (0, 0)),
            pl.BlockSpec((D, 1), lambda p: (0, 0)),
        ],
        out_specs=pl.BlockSpec((D, NP), lambda p: (p, 0)),
        out_shape=jax.ShapeDtypeStruct((FEAT, NP), jnp.float32),
    )(y1, nd2, ns2, W1, b1c)


def _d2_body(y_ref, nd_ref, w2_ref, b2_ref, wfc_ref, bfc_ref, o_ref):
    p = pl.program_id(0)
    y = y_ref[...] * nd_ref[...]
    h = lax.dot_general(w2_ref[...], y, (((0,), (0,)), ((), ())),
                        preferred_element_type=jnp.float32,
                        precision=lax.Precision.HIGHEST)
    h = h + b2_ref[...]
    h = jnp.where(h >= 0, h, 0.01 * h)
    mask = lax.broadcasted_iota(jnp.int32, (D, NP), 1) < N
    h = jnp.where(mask, h, 0.0)
    s = jnp.sum(h, axis=1, keepdims=True)
    contrib = lax.dot_general(s, wfc_ref[...], (((0,), (0,)), ((), ())),
                              preferred_element_type=jnp.float32,
                              precision=lax.Precision.HIGHEST)

    @pl.when(p == 0)
    def _():
        o_ref[...] = bfc_ref[...]

    o_ref[...] += contrib * (1.0 / N)


def _final_layer(y2, nd2, W2, b2c, W_fc_t, bfc2):
    return pl.pallas_call(
        _d2_body,
        grid=(PATCH,),
        in_specs=[
            pl.BlockSpec((D, NP), lambda p: (p, 0)),
            pl.BlockSpec((1, NP), lambda p: (0, 0)),
            pl.BlockSpec((D, D), lambda p: (0, 0)),
            pl.BlockSpec((D, 1), lambda p: (0, 0)),
            pl.BlockSpec((D, D), lambda p: (p, 0)),
            pl.BlockSpec((1, D), lambda p: (0, 0)),
        ],
        out_specs=pl.BlockSpec((1, D), lambda p: (0, 0)),
        out_shape=jax.ShapeDtypeStruct((1, D), jnp.float32),
    )(y2, nd2, W2, b2c, W_fc_t, bfc2)


def kernel(n_feat, edge_index, W1, b1, W2, b2, W_fc, b_fc):
    x0 = n_feat.reshape(N, FEAT)
    ns, nd, pk = _prep(edge_index.reshape(2 * E))
    ns2 = ns.reshape(1, NP)
    nd2 = nd.reshape(1, NP)
    xt = _transpose_scale(x0, ns2)
    y1 = _spmm(xt, pk)
    z1 = _mid_layer(y1, nd2, ns2, W1, b1.reshape(D, 1))
    y2 = _spmm(z1, pk)
    return _final_layer(y2, nd2, W2, b2.reshape(D, 1), W_fc.T, b_fc.reshape(1, D))


# speedup vs baseline: 9.1415x; 1.0849x over previous
import jax
import jax.numpy as jnp
from jax import lax
from jax.experimental import pallas as pl
from jax.experimental.pallas import tpu as pltpu
from jax.experimental.pallas import tpu_sc as plsc

N = 10000
NP = 10240
E = 40000
PATCH = 84
D = 32
FEAT = PATCH * D

NW = 32
ROWS_PW = FEAT // NW
FPP = 4
PASSES = ROWS_PW // FPP


def _vmesh():
    return plsc.VectorSubcoreMesh(core_axis_name="c", subcore_axis_name="s",
                                  num_cores=2, num_subcores=16)


def _prep_body(edge_ref, ns_ref, nd_ref, pk_ref, esrc, sbuf, dbuf, pbuf, hist):
    wid = lax.axis_index("s") * 2 + lax.axis_index("c")
    ones = jnp.full((16,), 1.0, jnp.float32)

    def hist_norm(row, out_hbm):
        pltpu.sync_copy(edge_ref.at[pl.ds(row * E, E)], esrc)

        def zero(i, _):
            hist[pl.ds(i * 16, 16)] = jnp.zeros((16,), jnp.float32)
            return 0
        lax.fori_loop(0, NP // 16, zero, 0)

        def acc(g, _):
            idx = esrc[pl.ds(g * 16, 16)]
            plsc.addupdate_scatter(hist, [idx], ones)
            return 0
        lax.fori_loop(0, E // 16, acc, 0)

        def norm(i, _):
            deg = hist[pl.ds(i * 16, 16)]
            m = jnp.maximum(deg, 1.0)
            bi = plsc.bitcast(m, jnp.int32)
            bi = jnp.int32(0x5F3759DF) - (bi >> 1)
            y = plsc.bitcast(bi, jnp.float32)
            for _ in range(4):
                y = y * (1.5 - 0.5 * m * y * y)
            hist[pl.ds(i * 16, 16)] = y
            return 0
        lax.fori_loop(0, NP // 16, norm, 0)
        pltpu.sync_copy(hist, out_hbm)

    @pl.when(wid == 0)
    def _():
        hist_norm(0, ns_ref)

    @pl.when(wid == 1)
    def _():
        hist_norm(1, nd_ref)

    @pl.when((wid >= 2) & (wid < 27))
    def _():
        off = (wid - 2) * (E // 25)
        pltpu.sync_copy(edge_ref.at[pl.ds(off, E // 25)], sbuf)
        pltpu.sync_copy(edge_ref.at[pl.ds(E + off, E // 25)], dbuf)

        def pk(g, _):
            s = sbuf[pl.ds(g * 16, 16)]
            d = dbuf[pl.ds(g * 16, 16)]
            pbuf[pl.ds(g * 16, 16)] = s | (d << 14)
            return 0
        lax.fori_loop(0, (E // 25) // 16, pk, 0)
        pltpu.sync_copy(pbuf, pk_ref.at[pl.ds(off, E // 25)])


_prep = pl.kernel(
    _prep_body,
    out_type=[
        jax.ShapeDtypeStruct((NP,), jnp.float32),
        jax.ShapeDtypeStruct((NP,), jnp.float32),
        jax.ShapeDtypeStruct((E,), jnp.int32),
    ],
    mesh=_vmesh(),
    scratch_types=[
        pltpu.VMEM((E,), jnp.int32),
        pltpu.VMEM((E // 25,), jnp.int32),
        pltpu.VMEM((E // 25,), jnp.int32),
        pltpu.VMEM((E // 25,), jnp.int32),
        pltpu.VMEM((NP,), jnp.float32),
    ],
    compiler_params=pltpu.CompilerParams(needs_layout_passes=False),
)


def _spmm_body(xt_ref, pk_ref, yt_ref, ebuf, x_buf, y_buf, xsem, ysem):
    wid = lax.axis_index("s") * 2 + lax.axis_index("c")
    pltpu.sync_copy(pk_ref, ebuf)
    base = wid * ROWS_PW
    mask14 = jnp.full((16,), 0x3FFF, jnp.int32)

    def one_pass(p, _):
        row0 = base + p * FPP
        xcp = pltpu.async_copy(xt_ref.at[pl.ds(row0, FPP)], x_buf, xsem)

        @pl.when(p > 0)
        def _():
            pltpu.make_async_copy(
                y_buf, yt_ref.at[pl.ds(row0 - FPP, FPP)], ysem).wait()

        @plsc.parallel_loop(0, NP // 16, unroll=4)
        def _zero(i):
            for r in range(FPP):
                y_buf[r, pl.ds(i * 16, 16)] = jnp.zeros((16,), jnp.float32)

        xcp.wait()

        @plsc.parallel_loop(0, E // 16, unroll=8)
        def _edge(g):
            pe = ebuf[pl.ds(g * 16, 16)]
            s = pe & mask14
            d = lax.shift_right_logical(pe, 14)
            for r in range(FPP):
                ridx = jnp.full((16,), r, jnp.int32)
                v = plsc.load_gather(x_buf, [ridx, s])
                plsc.addupdate_scatter(y_buf, [ridx, d], v)

        pltpu.async_copy(y_buf, yt_ref.at[pl.ds(row0, FPP)], ysem)
        return 0
    lax.fori_loop(0, PASSES, one_pass, 0)
    pltpu.make_async_copy(
        y_buf, yt_ref.at[pl.ds(base + (PASSES - 1) * FPP, FPP)], ysem).wait()


_spmm = pl.kernel(
    _spmm_body,
    out_type=jax.ShapeDtypeStruct((FEAT, NP), jnp.float32),
    mesh=_vmesh(),
    scratch_types=[
        pltpu.VMEM((E,), jnp.int32),
        pltpu.VMEM((FPP, NP), jnp.float32),
        pltpu.VMEM((FPP, NP), jnp.float32),
        pltpu.SemaphoreType.DMA,
        pltpu.SemaphoreType.DMA,
    ],
    compiler_params=pltpu.CompilerParams(needs_layout_passes=False),
)


def _tr_body(x_ref, ns_ref, o_ref):
    o_ref[...] = jnp.transpose(x_ref[...]) * ns_ref[...]


def _transpose_scale(x0, ns2):
    blk = 512
    return pl.pallas_call(
        _tr_body,
        grid=(NP // blk,),
        in_specs=[
            pl.BlockSpec((blk, FEAT), lambda g: (g, 0)),
            pl.BlockSpec((1, blk), lambda g: (0, g)),
        ],
        out_specs=pl.BlockSpec((FEAT, blk), lambda g: (0, g)),
        out_shape=jax.ShapeDtypeStruct((FEAT, NP), jnp.float32),
    )(x0, ns2)


def _d1_body(y_ref, nd_ref, ns_ref, w_ref, b_ref, o_ref):
    y = y_ref[...] * nd_ref[...]
    h = lax.dot_general(w_ref[...], y, (((0,), (0,)), ((), ())),
                        preferred_element_type=jnp.float32,
                        precision=lax.Precision.HIGHEST)
    h = h + b_ref[...]
    h = jnp.where(h >= 0, h, 0.01 * h)
    o_ref[...] = h * ns_ref[...]


def _mid_layer(y1, nd2, ns2, W1, b1c):
    return pl.pallas_call(
        _d1_body,
        grid=(PATCH,),
        in_specs=[
            pl.BlockSpec((D, NP), lambda p: (p, 0)),
            pl.BlockSpec((1, NP), lambda p: (0, 0)),
            pl.BlockSpec((1, NP), lambda p: (0, 0)),
            pl.BlockSpec((D, D), lambda p: (0, 0)),
            pl.BlockSpec((D, 1), lambda p: (0, 0)),
        ],
        out_specs=pl.BlockSpec((D, NP), lambda p: (p, 0)),
        out_shape=jax.ShapeDtypeStruct((FEAT, NP), jnp.float32),
    )(y1, nd2, ns2, W1, b1c)


def _d2_body(y_ref, nd_ref, w2_ref, b2_ref, wfc_ref, bfc_ref, o_ref):
    p = pl.program_id(0)
    y = y_ref[...] * nd_ref[...]
    h = lax.dot_general(w2_ref[...], y, (((0,), (0,)), ((), ())),
                        preferred_element_type=jnp.float32,
                        precision=lax.Precision.HIGHEST)
    h = h + b2_ref[...]
    h = jnp.where(h >= 0, h, 0.01 * h)
    mask = lax.broadcasted_iota(jnp.int32, (D, NP), 1) < N
    h = jnp.where(mask, h, 0.0)
    s = jnp.sum(h, axis=1, keepdims=True)
    contrib = lax.dot_general(s, wfc_ref[...], (((0,), (0,)), ((), ())),
                              preferred_element_type=jnp.float32,
                              precision=lax.Precision.HIGHEST)

    @pl.when(p == 0)
    def _():
        o_ref[...] = bfc_ref[...]

    o_ref[...] += contrib * (1.0 / N)


def _final_layer(y2, nd2, W2, b2c, W_fc_t, bfc2):
    return pl.pallas_call(
        _d2_body,
        grid=(PATCH,),
        in_specs=[
            pl.BlockSpec((D, NP), lambda p: (p, 0)),
            pl.BlockSpec((1, NP), lambda p: (0, 0)),
            pl.BlockSpec((D, D), lambda p: (0, 0)),
            pl.BlockSpec((D, 1), lambda p: (0, 0)),
            pl.BlockSpec((D, D), lambda p: (p, 0)),
            pl.BlockSpec((1, D), lambda p: (0, 0)),
        ],
        out_specs=pl.BlockSpec((1, D), lambda p: (0, 0)),
        out_shape=jax.ShapeDtypeStruct((1, D), jnp.float32),
    )(y2, nd2, W2, b2c, W_fc_t, bfc2)


def kernel(n_feat, edge_index, W1, b1, W2, b2, W_fc, b_fc):
    x0 = n_feat.reshape(N, FEAT)
    ns, nd, pk = _prep(edge_index.reshape(2 * E))
    ns2 = ns.reshape(1, NP)
    nd2 = nd.reshape(1, NP)
    xt = _transpose_scale(x0, ns2)
    y1 = _spmm(xt, pk)
    z1 = _mid_layer(y1, nd2, ns2, W1, b1.reshape(D, 1))
    y2 = _spmm(z1, pk)
    return _final_layer(y2, nd2, W2, b2.reshape(D, 1), W_fc.T, b_fc.reshape(1, D))


# speedup vs baseline: 9.1432x; 1.0002x over previous
import jax
import jax.numpy as jnp
from jax import lax
from jax.experimental import pallas as pl
from jax.experimental.pallas import tpu as pltpu
from jax.experimental.pallas import tpu_sc as plsc

N = 10000
NP = 10240
E = 40000
PATCH = 84
D = 32
FEAT = PATCH * D

NW = 32
ROWS_PW = FEAT // NW
FPP = 4
PASSES = ROWS_PW // FPP


def _vmesh():
    return plsc.VectorSubcoreMesh(core_axis_name="c", subcore_axis_name="s",
                                  num_cores=2, num_subcores=16)


def _prep_body(edge_ref, ns_ref, nd_ref, pk_ref, esrc, sbuf, dbuf, pbuf, hist):
    wid = lax.axis_index("s") * 2 + lax.axis_index("c")
    ones = jnp.full((16,), 1.0, jnp.float32)

    def hist_norm(row, out_hbm):
        pltpu.sync_copy(edge_ref.at[pl.ds(row * E, E)], esrc)

        @plsc.parallel_loop(0, NP // 16, unroll=4)
        def _zero(i):
            hist[pl.ds(i * 16, 16)] = jnp.zeros((16,), jnp.float32)

        @plsc.parallel_loop(0, E // 16, unroll=8)
        def _acc(g):
            idx = esrc[pl.ds(g * 16, 16)]
            plsc.addupdate_scatter(hist, [idx], ones)

        @plsc.parallel_loop(0, NP // 16, unroll=4)
        def _norm(i):
            deg = hist[pl.ds(i * 16, 16)]
            m = jnp.maximum(deg, 1.0)
            bi = plsc.bitcast(m, jnp.int32)
            bi = jnp.int32(0x5F3759DF) - (bi >> 1)
            y = plsc.bitcast(bi, jnp.float32)
            for _ in range(4):
                y = y * (1.5 - 0.5 * m * y * y)
            hist[pl.ds(i * 16, 16)] = y

        pltpu.sync_copy(hist, out_hbm)

    @pl.when(wid == 0)
    def _():
        hist_norm(0, ns_ref)

    @pl.when(wid == 1)
    def _():
        hist_norm(1, nd_ref)

    @pl.when((wid >= 2) & (wid < 27))
    def _():
        off = (wid - 2) * (E // 25)
        pltpu.sync_copy(edge_ref.at[pl.ds(off, E // 25)], sbuf)
        pltpu.sync_copy(edge_ref.at[pl.ds(E + off, E // 25)], dbuf)

        @plsc.parallel_loop(0, (E // 25) // 16, unroll=4)
        def _pk(g):
            s = sbuf[pl.ds(g * 16, 16)]
            d = dbuf[pl.ds(g * 16, 16)]
            pbuf[pl.ds(g * 16, 16)] = s | (d << 14)
        pltpu.sync_copy(pbuf, pk_ref.at[pl.ds(off, E // 25)])


_prep = pl.kernel(
    _prep_body,
    out_type=[
        jax.ShapeDtypeStruct((NP,), jnp.float32),
        jax.ShapeDtypeStruct((NP,), jnp.float32),
        jax.ShapeDtypeStruct((E,), jnp.int32),
    ],
    mesh=_vmesh(),
    scratch_types=[
        pltpu.VMEM((E,), jnp.int32),
        pltpu.VMEM((E // 25,), jnp.int32),
        pltpu.VMEM((E // 25,), jnp.int32),
        pltpu.VMEM((E // 25,), jnp.int32),
        pltpu.VMEM((NP,), jnp.float32),
    ],
    compiler_params=pltpu.CompilerParams(needs_layout_passes=False),
)


def _spmm_body(xt_ref, pk_ref, yt_ref, ebuf, x_buf, y_buf, xsem, ysem):
    wid = lax.axis_index("s") * 2 + lax.axis_index("c")
    pltpu.sync_copy(pk_ref, ebuf)
    base = wid * ROWS_PW
    mask14 = jnp.full((16,), 0x3FFF, jnp.int32)

    def one_pass(p, _):
        row0 = base + p * FPP
        xcp = pltpu.async_copy(xt_ref.at[pl.ds(row0, FPP)], x_buf, xsem)

        @pl.when(p > 0)
        def _():
            pltpu.make_async_copy(
                y_buf, yt_ref.at[pl.ds(row0 - FPP, FPP)], ysem).wait()

        @plsc.parallel_loop(0, NP // 16, unroll=4)
        def _zero(i):
            for r in range(FPP):
                y_buf[r, pl.ds(i * 16, 16)] = jnp.zeros((16,), jnp.float32)

        xcp.wait()

        @plsc.parallel_loop(0, E // 16, unroll=8)
        def _edge(g):
            pe = ebuf[pl.ds(g * 16, 16)]
            s = pe & mask14
            d = lax.shift_right_logical(pe, 14)
            for r in range(FPP):
                ridx = jnp.full((16,), r, jnp.int32)
                v = plsc.load_gather(x_buf, [ridx, s])
                plsc.addupdate_scatter(y_buf, [ridx, d], v)

        pltpu.async_copy(y_buf, yt_ref.at[pl.ds(row0, FPP)], ysem)
        return 0
    lax.fori_loop(0, PASSES, one_pass, 0)
    pltpu.make_async_copy(
        y_buf, yt_ref.at[pl.ds(base + (PASSES - 1) * FPP, FPP)], ysem).wait()


_spmm = pl.kernel(
    _spmm_body,
    out_type=jax.ShapeDtypeStruct((FEAT, NP), jnp.float32),
    mesh=_vmesh(),
    scratch_types=[
        pltpu.VMEM((E,), jnp.int32),
        pltpu.VMEM((FPP, NP), jnp.float32),
        pltpu.VMEM((FPP, NP), jnp.float32),
        pltpu.SemaphoreType.DMA,
        pltpu.SemaphoreType.DMA,
    ],
    compiler_params=pltpu.CompilerParams(needs_layout_passes=False),
)


def _tr_body(x_ref, ns_ref, o_ref):
    o_ref[...] = jnp.transpose(x_ref[...]) * ns_ref[...]


def _transpose_scale(x0, ns2):
    blk = 512
    return pl.pallas_call(
        _tr_body,
        grid=(NP // blk,),
        in_specs=[
            pl.BlockSpec((blk, FEAT), lambda g: (g, 0)),
            pl.BlockSpec((1, blk), lambda g: (0, g)),
        ],
        out_specs=pl.BlockSpec((FEAT, blk), lambda g: (0, g)),
        out_shape=jax.ShapeDtypeStruct((FEAT, NP), jnp.float32),
    )(x0, ns2)


def _d1_body(y_ref, nd_ref, ns_ref, w_ref, b_ref, o_ref):
    y = y_ref[...] * nd_ref[...]
    h = lax.dot_general(w_ref[...], y, (((0,), (0,)), ((), ())),
                        preferred_element_type=jnp.float32,
                        precision=lax.Precision.HIGHEST)
    h = h + b_ref[...]
    h = jnp.where(h >= 0, h, 0.01 * h)
    o_ref[...] = h * ns_ref[...]


def _mid_layer(y1, nd2, ns2, W1, b1c):
    return pl.pallas_call(
        _d1_body,
        grid=(PATCH,),
        in_specs=[
            pl.BlockSpec((D, NP), lambda p: (p, 0)),
            pl.BlockSpec((1, NP), lambda p: (0, 0)),
            pl.BlockSpec((1, NP), lambda p: (0, 0)),
            pl.BlockSpec((D, D), lambda p: (0, 0)),
            pl.BlockSpec((D, 1), lambda p: (0, 0)),
        ],
        out_specs=pl.BlockSpec((D, NP), lambda p: (p, 0)),
        out_shape=jax.ShapeDtypeStruct((FEAT, NP), jnp.float32),
    )(y1, nd2, ns2, W1, b1c)


def _d2_body(y_ref, nd_ref, w2_ref, b2_ref, wfc_ref, bfc_ref, o_ref):
    p = pl.program_id(0)
    y = y_ref[...] * nd_ref[...]
    h = lax.dot_general(w2_ref[...], y, (((0,), (0,)), ((), ())),
                        preferred_element_type=jnp.float32,
                        precision=lax.Precision.HIGHEST)
    h = h + b2_ref[...]
    h = jnp.where(h >= 0, h, 0.01 * h)
    mask = lax.broadcasted_iota(jnp.int32, (D, NP), 1) < N
    h = jnp.where(mask, h, 0.0)
    s = jnp.sum(h, axis=1, keepdims=True)
    contrib = lax.dot_general(s, wfc_ref[...], (((0,), (0,)), ((), ())),
                              preferred_element_type=jnp.float32,
                              precision=lax.Precision.HIGHEST)

    @pl.when(p == 0)
    def _():
        o_ref[...] = bfc_ref[...]

    o_ref[...] += contrib * (1.0 / N)


def _final_layer(y2, nd2, W2, b2c, W_fc_t, bfc2):
    return pl.pallas_call(
        _d2_body,
        grid=(PATCH,),
        in_specs=[
            pl.BlockSpec((D, NP), lambda p: (p, 0)),
            pl.BlockSpec((1, NP), lambda p: (0, 0)),
            pl.BlockSpec((D, D), lambda p: (0, 0)),
            pl.BlockSpec((D, 1), lambda p: (0, 0)),
            pl.BlockSpec((D, D), lambda p: (p, 0)),
            pl.BlockSpec((1, D), lambda p: (0, 0)),
        ],
        out_specs=pl.BlockSpec((1, D), lambda p: (0, 0)),
        out_shape=jax.ShapeDtypeStruct((1, D), jnp.float32),
    )(y2, nd2, W2, b2c, W_fc_t, bfc2)


def kernel(n_feat, edge_index, W1, b1, W2, b2, W_fc, b_fc):
    x0 = n_feat.reshape(N, FEAT)
    ns, nd, pk = _prep(edge_index.reshape(2 * E))
    ns2 = ns.reshape(1, NP)
    nd2 = nd.reshape(1, NP)
    xt = _transpose_scale(x0, ns2)
    y1 = _spmm(xt, pk)
    z1 = _mid_layer(y1, nd2, ns2, W1, b1.reshape(D, 1))
    y2 = _spmm(z1, pk)
    return _final_layer(y2, nd2, W2, b2.reshape(D, 1), W_fc.T, b_fc.reshape(1, D))
